# baseline (speedup 1.0000x reference)
import functools

import jax
import jax.numpy as jnp
from jax import lax
from jax.experimental import pallas as pl
from jax.experimental.pallas import tpu as pltpu
from jax.experimental.pallas import tpu_sc as plsc

E = 8
D = 1024
DFF = 512
SHARED_DFF = 1024
T = 2048
S = 2 * T
TBG = 256
NBP = S // TBG + (E - 1)
SP = NBP * TBG
MPAD = 64
TB = 256
NW = 32
TPW = T // NW

_DEF = lax.Precision.DEFAULT
_HI = lax.Precision.HIGHEST


def _dot(a, b, prec=_DEF):
    return lax.dot_general(a, b, (((1,), (1,)), ((), ())),
                           preferred_element_type=jnp.float32,
                           precision=prec)


def _glu(x, wg, wu, wd):
    g = _dot(x, wg)
    u = _dot(x, wu)
    h = (g * (1.0 / (1.0 + jnp.exp(-g)))) * u
    return _dot(h, wd)


def _gate_kernel(x_ref, gw_ref, dst_ref, wts_ref, meta_ref, xb_ref, ind_ref):
    x = x_ref[...]
    xb_ref[...] = x.astype(jnp.bfloat16)
    logits = _dot(x, gw_ref[...], _DEF)
    iota8 = lax.broadcasted_iota(jnp.int32, (T, E), 1)
    m1 = jnp.max(logits, axis=1, keepdims=True)
    a1 = jnp.min(jnp.where(logits == m1, iota8, E), axis=1, keepdims=True)
    l2 = jnp.where(iota8 == a1, -jnp.inf, logits)
    m2 = jnp.max(l2, axis=1, keepdims=True)
    a2 = jnp.min(jnp.where(l2 == m2, iota8, E), axis=1, keepdims=True)
    r = jnp.exp(m2 - m1)
    w1 = 1.0 / (1.0 + r)
    w2 = r / (1.0 + r)
    wts_ref[...] = jnp.concatenate([w1, w2], axis=1)

    ind = ((iota8 == a1) | (iota8 == a2)).astype(jnp.float32)
    ind_ref[...] = ind

    ctril = (lax.broadcasted_iota(jnp.int32, (T // E, T // E), 0)
             > lax.broadcasted_iota(jnp.int32, (T // E, T // E), 1)
             ).astype(jnp.float32)

    def body2(i, carry):
        sl = pl.ds(i * (T // E), T // E)
        ind_c = ind_ref[sl, :]
        rank_c = lax.dot_general(ctril, ind_c, (((1,), (0,)), ((), ())),
                                 preferred_element_type=jnp.float32,
                                 precision=_DEF) + carry
        ind_ref[sl, :] = rank_c
        return carry + jnp.sum(ind_c, axis=0, keepdims=True)

    counts_l = lax.fori_loop(0, E, body2, jnp.zeros((1, E), jnp.float32))

    pc_l = jnp.floor((counts_l + float(TBG - 1)) * (1.0 / TBG)) * float(TBG)
    u8 = (lax.broadcasted_iota(jnp.int32, (E, E), 0)
          < lax.broadcasted_iota(jnp.int32, (E, E), 1)).astype(jnp.float32)
    poffs_l = lax.dot_general(pc_l, u8, (((1,), (0,)), ((), ())),
                              preferred_element_type=jnp.float32,
                              precision=_HI)

    rank = ind_ref[...]
    val = poffs_l + rank
    d1 = jnp.sum(jnp.where(iota8 == a1, val, 0.0), axis=1, keepdims=True)
    d2 = jnp.sum(jnp.where(iota8 == a2, val, 0.0), axis=1, keepdims=True)
    dst_ref[:, 0, :] = d1.astype(jnp.int32).reshape(NW, TPW)
    dst_ref[:, 1, :] = d2.astype(jnp.int32).reshape(NW, TPW)

    ib = lax.broadcasted_iota(jnp.int32, (1, MPAD), 1)
    bval = (ib * TBG).astype(jnp.float32)

    ones_t = jnp.ones((T, 1), jnp.float32)
    counts_s = lax.dot_general(ind, ones_t, (((0,), (0,)), ((), ())),
                               preferred_element_type=jnp.float32,
                               precision=_HI)
    pc_s = jnp.floor((counts_s + float(TBG - 1)) * (1.0 / TBG)) * float(TBG)
    l8 = (lax.broadcasted_iota(jnp.int32, (E, E), 1)
          < lax.broadcasted_iota(jnp.int32, (E, E), 0)).astype(jnp.float32)
    poffs_s = lax.dot_general(l8, pc_s, (((1,), (0,)), ((), ())),
                              preferred_element_type=jnp.float32,
                              precision=_HI)
    js = lax.broadcasted_iota(jnp.int32, (E, 1), 0)

    ebk = jnp.sum(jnp.where((js >= 1) & (poffs_s <= bval), 1, 0),
                  axis=0, keepdims=True)
    ptot = jnp.sum(pc_s, axis=0, keepdims=True)
    lae = jnp.max(jnp.where(pc_s > 0, js, 0), axis=0, keepdims=True)
    ebk = jnp.where(bval < ptot, ebk, lae)
    rid = jnp.sum(jnp.where((pc_s > 0) & (js >= 1) & (poffs_s <= bval), 1, 0),
                  axis=0, keepdims=True)
    meta_ref[0:1, :] = ebk
    meta_ref[1:2, :] = rid & 1
    meta_ref[2:8, :] = jnp.zeros((6, MPAD), jnp.int32)


def _gate(x, gate_weight):
    return pl.pallas_call(
        _gate_kernel,
        out_shape=(
            jax.ShapeDtypeStruct((NW, 2, TPW), jnp.int32),
            jax.ShapeDtypeStruct((T, 2), jnp.float32),
            jax.ShapeDtypeStruct((E, MPAD), jnp.int32),
            jax.ShapeDtypeStruct((T, D), jnp.bfloat16),
        ),
        scratch_shapes=[pltpu.VMEM((T, E), jnp.float32)],
    )(x, gate_weight)


def _sc_scatter(x, dst3):
    mesh = plsc.VectorSubcoreMesh(core_axis_name="c", subcore_axis_name="s")

    @functools.partial(
        pl.kernel,
        out_type=jax.ShapeDtypeStruct((SP, D // 2), jnp.int32),
        mesh=mesh,
        scratch_types=[
            pltpu.VMEM((2, TPW), jnp.int32),
            pltpu.VMEM((TPW, D // 2), jnp.int32),
            pltpu.SemaphoreType.DMA,
            pltpu.SemaphoreType.DMA,
        ],
    )
    def k(x_hbm, dst_hbm, xs_hbm, idx_v, rows_v, sem0, sem1):
        wid = lax.axis_index("s") * 2 + lax.axis_index("c")
        base = wid * TPW
        pltpu.sync_copy(dst_hbm.at[wid], idx_v)
        pltpu.sync_copy(x_hbm.at[pl.ds(base, TPW)], rows_v)
        h0 = pltpu.async_copy(rows_v, xs_hbm.at[idx_v.at[0]], sem0)
        h1 = pltpu.async_copy(rows_v, xs_hbm.at[idx_v.at[1]], sem1)
        h0.wait()
        h1.wait()

    return k(x, dst3)


def _grouped_kernel(meta_ref, xs_ref, wg_hbm, wu_hbm, wd_hbm, ys_ref,
                    wg_v, wu_v, wd_v, sems):
    b = pl.program_id(0)
    e = meta_ref[0, b]
    prev_e = meta_ref[0, jnp.maximum(b - 1, 0)]
    is_new = jnp.logical_or(b == 0, e != prev_e)
    ne = meta_ref[0, jnp.minimum(b + 1, NBP - 1)]
    next_new = jnp.logical_and(b + 1 < NBP, ne != e)
    slot = meta_ref[1, b]
    nslot = meta_ref[1, jnp.minimum(b + 1, NBP - 1)]

    def _copies(ee, sl):
        return (
            pltpu.make_async_copy(wg_hbm.at[ee], wg_v.at[sl], sems.at[0, sl]),
            pltpu.make_async_copy(wu_hbm.at[ee], wu_v.at[sl], sems.at[1, sl]),
            pltpu.make_async_copy(wd_hbm.at[ee], wd_v.at[sl], sems.at[2, sl]),
        )

    @pl.when(b == 0)
    def _():
        for c in _copies(e, slot):
            c.start()

    @pl.when(is_new)
    def _():
        for c in _copies(e, slot):
            c.wait()

    @pl.when(next_new)
    def _():
        for c in _copies(ne, nslot):
            c.start()

    x32 = xs_ref[...].astype(jnp.float32)
    ys_ref[...] = _glu(x32, wg_v[slot], wu_v[slot], wd_v[slot]
                       ).astype(jnp.bfloat16)


def _grouped(meta_t, xs, Wg, Wu, Wd):
    grid_spec = pltpu.PrefetchScalarGridSpec(
        num_scalar_prefetch=1,
        grid=(NBP,),
        in_specs=[
            pl.BlockSpec((TBG, D), lambda b, m: (b, 0)),
            pl.BlockSpec(memory_space=pl.ANY),
            pl.BlockSpec(memory_space=pl.ANY),
            pl.BlockSpec(memory_space=pl.ANY),
        ],
        out_specs=pl.BlockSpec((TBG, D), lambda b, m: (b, 0)),
        scratch_shapes=[
            pltpu.VMEM((2, DFF, D), jnp.float32),
            pltpu.VMEM((2, DFF, D), jnp.float32),
            pltpu.VMEM((2, D, DFF), jnp.float32),
            pltpu.SemaphoreType.DMA((3, 2)),
        ],
    )
    return pl.pallas_call(
        _grouped_kernel,
        grid_spec=grid_spec,
        out_shape=jax.ShapeDtypeStruct((SP, D), jnp.bfloat16),
        compiler_params=pltpu.CompilerParams(
            dimension_semantics=("arbitrary",)),
    )(meta_t, xs, Wg, Wu, Wd)


def _sc_gather(ys, dst3):
    mesh = plsc.VectorSubcoreMesh(core_axis_name="c", subcore_axis_name="s")

    @functools.partial(
        pl.kernel,
        out_type=(jax.ShapeDtypeStruct((T, D // 2), jnp.int32),
                  jax.ShapeDtypeStruct((T, D // 2), jnp.int32)),
        mesh=mesh,
        scratch_types=[
            pltpu.VMEM((2, TPW), jnp.int32),
            pltpu.VMEM((TPW, D // 2), jnp.int32),
            pltpu.SemaphoreType.DMA,
        ],
    )
    def k(ys_hbm, dst_hbm, y0_hbm, y1_hbm, idx_v, buf_v, sem):
        wid = lax.axis_index("s") * 2 + lax.axis_index("c")
        base = wid * TPW
        pltpu.sync_copy(dst_hbm.at[wid], idx_v)
        pltpu.async_copy(ys_hbm.at[idx_v.at[0]], buf_v, sem).wait()
        pltpu.sync_copy(buf_v, y0_hbm.at[pl.ds(base, TPW)])
        pltpu.async_copy(ys_hbm.at[idx_v.at[1]], buf_v, sem).wait()
        pltpu.sync_copy(buf_v, y1_hbm.at[pl.ds(base, TPW)])

    return k(ys, dst3)


def _shared_kernel(x_ref, swg_ref, swu_ref, swd_ref, out_ref):
    out_ref[...] = _glu(x_ref[...], swg_ref[...], swu_ref[...], swd_ref[...])


def _shared(x, sWg, sWu, sWd):
    return pl.pallas_call(
        _shared_kernel,
        grid=(T // TB,),
        in_specs=[
            pl.BlockSpec((TB, D), lambda i: (i, 0)),
            pl.BlockSpec((SHARED_DFF, D), lambda i: (0, 0)),
            pl.BlockSpec((SHARED_DFF, D), lambda i: (0, 0)),
            pl.BlockSpec((D, SHARED_DFF), lambda i: (0, 0)),
        ],
        out_specs=pl.BlockSpec((TB, D), lambda i: (i, 0)),
        out_shape=jax.ShapeDtypeStruct((T, D), jnp.float32),
    )(x, sWg, sWu, sWd)


def _final_kernel(ysh_ref, y0_ref, y1_ref, w_ref, out_ref):
    w = w_ref[...]
    out_ref[...] = (ysh_ref[...] + w[:, 0:1] * y0_ref[...]
                    + w[:, 1:2] * y1_ref[...])


def _final(y_sh, y0, y1, wts):
    fb = 512
    return pl.pallas_call(
        _final_kernel,
        grid=(T // fb,),
        in_specs=[
            pl.BlockSpec((fb, D), lambda i: (i, 0)),
            pl.BlockSpec((fb, D), lambda i: (i, 0)),
            pl.BlockSpec((fb, D), lambda i: (i, 0)),
            pl.BlockSpec((fb, 2), lambda i: (i, 0)),
        ],
        out_specs=pl.BlockSpec((fb, D), lambda i: (i, 0)),
        out_shape=jax.ShapeDtypeStruct((T, D), jnp.float32),
    )(y_sh, y0, y1, wts)


def _as_i32(a):
    n = a.shape[0]
    return lax.bitcast_convert_type(a.reshape(n, -1, 2), jnp.int32)


def _as_bf16(a):
    n = a.shape[0]
    return lax.bitcast_convert_type(a, jnp.bfloat16).reshape(n, -1)


def kernel(hidden_states, gate_weight, Wg, Wu, Wd, sWg, sWu, sWd):
    x = hidden_states
    dst3, wts, meta, xb = _gate(x, gate_weight)
    xs = _as_bf16(_sc_scatter(_as_i32(xb), dst3))
    y_sh = _shared(x, sWg, sWu, sWd)
    ys = _grouped(meta, xs, Wg, Wu, Wd)
    y0, y1 = _sc_gather(_as_i32(ys), dst3)
    return _final(y_sh, _as_bf16(y0), _as_bf16(y1), wts)


# speedup vs baseline: 4.2535x; 4.2535x over previous
import functools

import jax
import jax.numpy as jnp
from jax import lax
from jax.experimental import pallas as pl
from jax.experimental.pallas import tpu as pltpu
from jax.experimental.pallas import tpu_sc as plsc

E = 8
D = 1024
DFF = 512
SHARED_DFF = 1024
T = 2048
S = 2 * T
TBG = 256
NBP = S // TBG + (E - 1)
SP = NBP * TBG
MPAD = 64
TB = 256
NW = 32
TPW = T // NW

_DEF = lax.Precision.DEFAULT
_HI = lax.Precision.HIGHEST


def _dot(a, b, prec=_DEF):
    return lax.dot_general(a, b, (((1,), (1,)), ((), ())),
                           preferred_element_type=jnp.float32,
                           precision=prec)


def _glu(x, wg, wu, wd):
    g = _dot(x, wg)
    u = _dot(x, wu)
    h = (g * (1.0 / (1.0 + jnp.exp(-g)))) * u
    return _dot(h, wd)


def _gate_kernel(x_ref, gw_ref, dst_ref, wts_ref, meta_ref, ind_ref):
    x = x_ref[...]
    logits = _dot(x, gw_ref[...], _DEF)
    iota8 = lax.broadcasted_iota(jnp.int32, (T, E), 1)
    m1 = jnp.max(logits, axis=1, keepdims=True)
    a1 = jnp.min(jnp.where(logits == m1, iota8, E), axis=1, keepdims=True)
    l2 = jnp.where(iota8 == a1, -jnp.inf, logits)
    m2 = jnp.max(l2, axis=1, keepdims=True)
    a2 = jnp.min(jnp.where(l2 == m2, iota8, E), axis=1, keepdims=True)
    r = jnp.exp(m2 - m1)
    w1 = 1.0 / (1.0 + r)
    w2 = r / (1.0 + r)
    wts_ref[...] = jnp.concatenate([w1, w2], axis=1)

    ind = ((iota8 == a1) | (iota8 == a2)).astype(jnp.float32)
    ind_ref[...] = ind

    ctril = (lax.broadcasted_iota(jnp.int32, (T // E, T // E), 0)
             > lax.broadcasted_iota(jnp.int32, (T // E, T // E), 1)
             ).astype(jnp.float32)

    def body2(i, carry):
        sl = pl.ds(i * (T // E), T // E)
        ind_c = ind_ref[sl, :]
        rank_c = lax.dot_general(ctril, ind_c, (((1,), (0,)), ((), ())),
                                 preferred_element_type=jnp.float32,
                                 precision=_DEF) + carry
        ind_ref[sl, :] = rank_c
        return carry + jnp.sum(ind_c, axis=0, keepdims=True)

    counts_l = lax.fori_loop(0, E, body2, jnp.zeros((1, E), jnp.float32))

    pc_l = jnp.floor((counts_l + float(TBG - 1)) * (1.0 / TBG)) * float(TBG)
    u8 = (lax.broadcasted_iota(jnp.int32, (E, E), 0)
          < lax.broadcasted_iota(jnp.int32, (E, E), 1)).astype(jnp.float32)
    poffs_l = lax.dot_general(pc_l, u8, (((1,), (0,)), ((), ())),
                              preferred_element_type=jnp.float32,
                              precision=_HI)

    rank = ind_ref[...]
    val = poffs_l + rank
    d1 = jnp.sum(jnp.where(iota8 == a1, val, 0.0), axis=1, keepdims=True)
    d2 = jnp.sum(jnp.where(iota8 == a2, val, 0.0), axis=1, keepdims=True)
    dst_ref[:, 0, :] = d1.astype(jnp.int32).reshape(NW, TPW)
    dst_ref[:, 1, :] = d2.astype(jnp.int32).reshape(NW, TPW)

    ib = lax.broadcasted_iota(jnp.int32, (1, MPAD), 1)
    bval = (ib * TBG).astype(jnp.float32)

    ones_t = jnp.ones((T, 1), jnp.float32)
    counts_s = lax.dot_general(ind, ones_t, (((0,), (0,)), ((), ())),
                               preferred_element_type=jnp.float32,
                               precision=_HI)
    pc_s = jnp.floor((counts_s + float(TBG - 1)) * (1.0 / TBG)) * float(TBG)
    l8 = (lax.broadcasted_iota(jnp.int32, (E, E), 1)
          < lax.broadcasted_iota(jnp.int32, (E, E), 0)).astype(jnp.float32)
    poffs_s = lax.dot_general(l8, pc_s, (((1,), (0,)), ((), ())),
                              preferred_element_type=jnp.float32,
                              precision=_HI)
    js = lax.broadcasted_iota(jnp.int32, (E, 1), 0)

    ebk = jnp.sum(jnp.where((js >= 1) & (poffs_s <= bval), 1, 0),
                  axis=0, keepdims=True)
    ptot = jnp.sum(pc_s, axis=0, keepdims=True)
    lae = jnp.max(jnp.where(pc_s > 0, js, 0), axis=0, keepdims=True)
    ebk = jnp.where(bval < ptot, ebk, lae)
    rid = jnp.sum(jnp.where((pc_s > 0) & (js >= 1) & (poffs_s <= bval), 1, 0),
                  axis=0, keepdims=True)
    meta_ref[0:1, :] = ebk
    meta_ref[1:2, :] = rid & 1
    meta_ref[2:8, :] = jnp.zeros((6, MPAD), jnp.int32)


def _gate(x, gate_weight):
    return pl.pallas_call(
        _gate_kernel,
        out_shape=(
            jax.ShapeDtypeStruct((NW, 2, TPW), jnp.int32),
            jax.ShapeDtypeStruct((T, 2), jnp.float32),
            jax.ShapeDtypeStruct((E, MPAD), jnp.int32),
        ),
        scratch_shapes=[pltpu.VMEM((T, E), jnp.float32)],
    )(x, gate_weight)


def _sc_scatter(x, dst3):
    mesh = plsc.VectorSubcoreMesh(core_axis_name="c", subcore_axis_name="s")

    @functools.partial(
        pl.kernel,
        out_type=jax.ShapeDtypeStruct((SP, D), jnp.float32),
        mesh=mesh,
        scratch_types=[
            pltpu.VMEM((2, TPW), jnp.int32),
            pltpu.VMEM((TPW, D), jnp.float32),
            pltpu.SemaphoreType.DMA,
            pltpu.SemaphoreType.DMA,
        ],
    )
    def k(x_hbm, dst_hbm, xs_hbm, idx_v, rows_v, sem0, sem1):
        wid = lax.axis_index("s") * 2 + lax.axis_index("c")
        base = wid * TPW
        pltpu.sync_copy(dst_hbm.at[wid], idx_v)
        pltpu.sync_copy(x_hbm.at[pl.ds(base, TPW)], rows_v)
        h0 = pltpu.async_copy(rows_v, xs_hbm.at[idx_v.at[0]], sem0)
        h1 = pltpu.async_copy(rows_v, xs_hbm.at[idx_v.at[1]], sem1)
        h0.wait()
        h1.wait()

    return k(x, dst3)


def _grouped_kernel(meta_ref, xs_ref, wg_ref, wu_ref, wd_ref, ys_ref):
    ys_ref[...] = _glu(xs_ref[...], wg_ref[0], wu_ref[0], wd_ref[0])


def _grouped(meta_t, xs, Wg, Wu, Wd):
    grid_spec = pltpu.PrefetchScalarGridSpec(
        num_scalar_prefetch=1,
        grid=(NBP,),
        in_specs=[
            pl.BlockSpec((TBG, D), lambda b, m: (b, 0)),
            pl.BlockSpec((1, DFF, D), lambda b, m: (m[0, b], 0, 0)),
            pl.BlockSpec((1, DFF, D), lambda b, m: (m[0, b], 0, 0)),
            pl.BlockSpec((1, D, DFF), lambda b, m: (m[0, b], 0, 0)),
        ],
        out_specs=pl.BlockSpec((TBG, D), lambda b, m: (b, 0)),
    )
    return pl.pallas_call(
        _grouped_kernel,
        grid_spec=grid_spec,
        out_shape=jax.ShapeDtypeStruct((SP, D), jnp.float32),
        compiler_params=pltpu.CompilerParams(
            dimension_semantics=("arbitrary",)),
    )(meta_t, xs, Wg, Wu, Wd)


def _sc_gather(ys, dst3):
    mesh = plsc.VectorSubcoreMesh(core_axis_name="c", subcore_axis_name="s")

    @functools.partial(
        pl.kernel,
        out_type=(jax.ShapeDtypeStruct((T, D), jnp.float32),
                  jax.ShapeDtypeStruct((T, D), jnp.float32)),
        mesh=mesh,
        scratch_types=[
            pltpu.VMEM((2, TPW), jnp.int32),
            pltpu.VMEM((TPW, D), jnp.float32),
            pltpu.SemaphoreType.DMA,
        ],
    )
    def k(ys_hbm, dst_hbm, y0_hbm, y1_hbm, idx_v, buf_v, sem):
        wid = lax.axis_index("s") * 2 + lax.axis_index("c")
        base = wid * TPW
        pltpu.sync_copy(dst_hbm.at[wid], idx_v)
        pltpu.async_copy(ys_hbm.at[idx_v.at[0]], buf_v, sem).wait()
        pltpu.sync_copy(buf_v, y0_hbm.at[pl.ds(base, TPW)])
        pltpu.async_copy(ys_hbm.at[idx_v.at[1]], buf_v, sem).wait()
        pltpu.sync_copy(buf_v, y1_hbm.at[pl.ds(base, TPW)])

    return k(ys, dst3)


def _shared_kernel(x_ref, swg_ref, swu_ref, swd_ref, out_ref):
    out_ref[...] = _glu(x_ref[...], swg_ref[...], swu_ref[...], swd_ref[...])


def _shared(x, sWg, sWu, sWd):
    return pl.pallas_call(
        _shared_kernel,
        grid=(T // TB,),
        in_specs=[
            pl.BlockSpec((TB, D), lambda i: (i, 0)),
            pl.BlockSpec((SHARED_DFF, D), lambda i: (0, 0)),
            pl.BlockSpec((SHARED_DFF, D), lambda i: (0, 0)),
            pl.BlockSpec((D, SHARED_DFF), lambda i: (0, 0)),
        ],
        out_specs=pl.BlockSpec((TB, D), lambda i: (i, 0)),
        out_shape=jax.ShapeDtypeStruct((T, D), jnp.float32),
    )(x, sWg, sWu, sWd)


def _final_kernel(ysh_ref, y0_ref, y1_ref, w_ref, out_ref):
    w = w_ref[...]
    out_ref[...] = (ysh_ref[...] + w[:, 0:1] * y0_ref[...]
                    + w[:, 1:2] * y1_ref[...])


def _final(y_sh, y0, y1, wts):
    fb = 512
    return pl.pallas_call(
        _final_kernel,
        grid=(T // fb,),
        in_specs=[
            pl.BlockSpec((fb, D), lambda i: (i, 0)),
            pl.BlockSpec((fb, D), lambda i: (i, 0)),
            pl.BlockSpec((fb, D), lambda i: (i, 0)),
            pl.BlockSpec((fb, 2), lambda i: (i, 0)),
        ],
        out_specs=pl.BlockSpec((fb, D), lambda i: (i, 0)),
        out_shape=jax.ShapeDtypeStruct((T, D), jnp.float32),
    )(y_sh, y0, y1, wts)


def kernel(hidden_states, gate_weight, Wg, Wu, Wd, sWg, sWu, sWd):
    x = hidden_states
    dst3, wts, meta = _gate(x, gate_weight)
    xs = _sc_scatter(x, dst3)
    y_sh = _shared(x, sWg, sWu, sWd)
    ys = _grouped(meta, xs, Wg, Wu, Wd)
    y0, y1 = _sc_gather(ys, dst3)
    return _final(y_sh, y0, y1, wts)


# speedup vs baseline: 4.3623x; 1.0256x over previous
import functools

import jax
import jax.numpy as jnp
from jax import lax
from jax.experimental import pallas as pl
from jax.experimental.pallas import tpu as pltpu
from jax.experimental.pallas import tpu_sc as plsc

E = 8
D = 1024
DFF = 512
SHARED_DFF = 1024
T = 2048
S = 2 * T
TBG = 256
NBG = S // TBG
NSEG = NBG + E - 1
MPAD = 64
TB = 256
NW = 32
TPW = T // NW

_DEF = lax.Precision.DEFAULT
_HI = lax.Precision.HIGHEST


def _dot(a, b, prec=_DEF):
    return lax.dot_general(a, b, (((1,), (1,)), ((), ())),
                           preferred_element_type=jnp.float32,
                           precision=prec)


def _glu(x, wg, wu, wd):
    g = _dot(x, wg)
    u = _dot(x, wu)
    h = (g * (1.0 / (1.0 + jnp.exp(-g)))) * u
    return _dot(h, wd)


def _gate_kernel(x_ref, gw_ref, dst_ref, wts_ref, meta_ref, ind_ref):
    x = x_ref[...]
    logits = _dot(x, gw_ref[...], _DEF)
    iota8 = lax.broadcasted_iota(jnp.int32, (T, E), 1)
    m1 = jnp.max(logits, axis=1, keepdims=True)
    a1 = jnp.min(jnp.where(logits == m1, iota8, E), axis=1, keepdims=True)
    l2 = jnp.where(iota8 == a1, -jnp.inf, logits)
    m2 = jnp.max(l2, axis=1, keepdims=True)
    a2 = jnp.min(jnp.where(l2 == m2, iota8, E), axis=1, keepdims=True)
    r = jnp.exp(m2 - m1)
    w1 = 1.0 / (1.0 + r)
    w2 = r / (1.0 + r)
    wts_ref[...] = jnp.concatenate([w1, w2], axis=1)

    ind = ((iota8 == a1) | (iota8 == a2)).astype(jnp.float32)
    ind_ref[...] = ind

    ctril = (lax.broadcasted_iota(jnp.int32, (T // E, T // E), 0)
             > lax.broadcasted_iota(jnp.int32, (T // E, T // E), 1)
             ).astype(jnp.float32)

    def body2(i, carry):
        sl = pl.ds(i * (T // E), T // E)
        ind_c = ind_ref[sl, :]
        rank_c = lax.dot_general(ctril, ind_c, (((1,), (0,)), ((), ())),
                                 preferred_element_type=jnp.float32,
                                 precision=_DEF) + carry
        ind_ref[sl, :] = rank_c
        return carry + jnp.sum(ind_c, axis=0, keepdims=True)

    counts_l = lax.fori_loop(0, E, body2, jnp.zeros((1, E), jnp.float32))

    u8 = (lax.broadcasted_iota(jnp.int32, (E, E), 0)
          < lax.broadcasted_iota(jnp.int32, (E, E), 1)).astype(jnp.float32)
    offs_l = lax.dot_general(counts_l, u8, (((1,), (0,)), ((), ())),
                             preferred_element_type=jnp.float32,
                             precision=_HI)

    rank = ind_ref[...]
    val = offs_l + rank
    d1 = jnp.sum(jnp.where(iota8 == a1, val, 0.0), axis=1, keepdims=True)
    d2 = jnp.sum(jnp.where(iota8 == a2, val, 0.0), axis=1, keepdims=True)
    dst_ref[:, 0, :] = d1.astype(jnp.int32).reshape(NW, TPW)
    dst_ref[:, 1, :] = d2.astype(jnp.int32).reshape(NW, TPW)

    jl = lax.broadcasted_iota(jnp.int32, (1, E), 1)
    ib = lax.broadcasted_iota(jnp.int32, (1, MPAD), 1)
    bval = (ib * TBG).astype(jnp.float32)
    valid_b = ib < NBG
    ib_s = lax.broadcasted_iota(jnp.int32, (MPAD, 1), 0)
    bval_s = (ib_s * TBG).astype(jnp.float32)
    valid_b_s = ib_s < NBG
    ip_l = lax.broadcasted_iota(jnp.int32, (1, MPAD), 1)

    ones_t = jnp.ones((T, 1), jnp.float32)
    counts_s = lax.dot_general(ind, ones_t, (((0,), (0,)), ((), ())),
                               preferred_element_type=jnp.float32,
                               precision=_HI)
    l8 = (lax.broadcasted_iota(jnp.int32, (E, E), 1)
          < lax.broadcasted_iota(jnp.int32, (E, E), 0)).astype(jnp.float32)
    offs_s = lax.dot_general(l8, counts_s, (((1,), (0,)), ((), ())),
                             preferred_element_type=jnp.float32,
                             precision=_HI)
    js = lax.broadcasted_iota(jnp.int32, (E, 1), 0)

    pos_b_s = ib_s + jnp.sum(jnp.where((jl >= 1) & (offs_l < bval_s), 1, 0),
                             axis=1, keepdims=True)
    pos_j_s = (js - 1) + jnp.sum(
        jnp.where(valid_b & (bval <= offs_s), 1, 0), axis=1,
        keepdims=True)

    def at_slot(q):
        t_b = jnp.sum(jnp.where((pos_b_s == q) & valid_b_s, bval_s, 0.0),
                      axis=0, keepdims=True)
        t_j = jnp.sum(jnp.where((pos_j_s == q) & (js >= 1), offs_s, 0.0),
                      axis=0, keepdims=True)
        return t_b + t_j

    seg_start = at_slot(ip_l)
    seg_end = at_slot(ip_l + 1) + jnp.where(ip_l == NSEG - 1, float(S), 0.0)
    expert = jnp.sum(jnp.where((js >= 1) & (offs_s <= seg_start), 1, 0),
                     axis=0, keepdims=True)
    init = jnp.sum(jnp.where((pos_b_s == ip_l) & valid_b_s, 1, 0),
                   axis=0, keepdims=True)
    seg_start_i = seg_start.astype(jnp.int32)
    block = seg_start_i // TBG

    meta_ref[0:1, :] = seg_start_i
    meta_ref[1:2, :] = seg_end.astype(jnp.int32)
    meta_ref[2:3, :] = expert
    meta_ref[3:4, :] = block
    meta_ref[4:5, :] = init
    meta_ref[5:8, :] = jnp.zeros((3, MPAD), jnp.int32)


def _gate(x, gate_weight):
    return pl.pallas_call(
        _gate_kernel,
        out_shape=(
            jax.ShapeDtypeStruct((NW, 2, TPW), jnp.int32),
            jax.ShapeDtypeStruct((T, 2), jnp.float32),
            jax.ShapeDtypeStruct((E, MPAD), jnp.int32),
        ),
        scratch_shapes=[pltpu.VMEM((T, E), jnp.float32)],
    )(x, gate_weight)


def _sc_scatter(x, dst3):
    mesh = plsc.VectorSubcoreMesh(core_axis_name="c", subcore_axis_name="s")

    @functools.partial(
        pl.kernel,
        out_type=jax.ShapeDtypeStruct((S, D), jnp.float32),
        mesh=mesh,
        scratch_types=[
            pltpu.VMEM((2, TPW), jnp.int32),
            pltpu.VMEM((TPW, D), jnp.float32),
            pltpu.SemaphoreType.DMA,
            pltpu.SemaphoreType.DMA,
        ],
    )
    def k(x_hbm, dst_hbm, xs_hbm, idx_v, rows_v, sem0, sem1):
        wid = lax.axis_index("s") * 2 + lax.axis_index("c")
        base = wid * TPW
        pltpu.sync_copy(dst_hbm.at[wid], idx_v)
        pltpu.sync_copy(x_hbm.at[pl.ds(base, TPW)], rows_v)
        h0 = pltpu.async_copy(rows_v, xs_hbm.at[idx_v.at[0]], sem0)
        h1 = pltpu.async_copy(rows_v, xs_hbm.at[idx_v.at[1]], sem1)
        h0.wait()
        h1.wait()

    return k(x, dst3)


def _grouped_kernel(meta_ref, xs_ref, wg_ref, wu_ref, wd_ref, ys_ref):
    s = pl.program_id(0)
    blk = meta_ref[3, s]
    lo = meta_ref[0, s] - blk * TBG
    hi = meta_ref[1, s] - blk * TBG
    y = _glu(xs_ref[...], wg_ref[0], wu_ref[0], wd_ref[0])
    rows = lax.broadcasted_iota(jnp.int32, (TBG, 1), 0)
    contrib = jnp.where((rows >= lo) & (rows < hi), y, 0.0)

    @pl.when(meta_ref[4, s] == 1)
    def _():
        ys_ref[...] = contrib

    @pl.when(meta_ref[4, s] == 0)
    def _():
        ys_ref[...] += contrib


def _grouped(meta_t, xs, Wg, Wu, Wd):
    grid_spec = pltpu.PrefetchScalarGridSpec(
        num_scalar_prefetch=1,
        grid=(NSEG,),
        in_specs=[
            pl.BlockSpec((TBG, D), lambda s, m: (m[3, s], 0)),
            pl.BlockSpec((1, DFF, D), lambda s, m: (m[2, s], 0, 0)),
            pl.BlockSpec((1, DFF, D), lambda s, m: (m[2, s], 0, 0)),
            pl.BlockSpec((1, D, DFF), lambda s, m: (m[2, s], 0, 0)),
        ],
        out_specs=pl.BlockSpec((TBG, D), lambda s, m: (m[3, s], 0)),
    )
    return pl.pallas_call(
        _grouped_kernel,
        grid_spec=grid_spec,
        out_shape=jax.ShapeDtypeStruct((S, D), jnp.float32),
        compiler_params=pltpu.CompilerParams(
            dimension_semantics=("arbitrary",)),
    )(meta_t, xs, Wg, Wu, Wd)


def _sc_gather(ys, dst3):
    mesh = plsc.VectorSubcoreMesh(core_axis_name="c", subcore_axis_name="s")

    @functools.partial(
        pl.kernel,
        out_type=(jax.ShapeDtypeStruct((T, D), jnp.float32),
                  jax.ShapeDtypeStruct((T, D), jnp.float32)),
        mesh=mesh,
        scratch_types=[
            pltpu.VMEM((2, TPW), jnp.int32),
            pltpu.VMEM((TPW, D), jnp.float32),
            pltpu.SemaphoreType.DMA,
        ],
    )
    def k(ys_hbm, dst_hbm, y0_hbm, y1_hbm, idx_v, buf_v, sem):
        wid = lax.axis_index("s") * 2 + lax.axis_index("c")
        base = wid * TPW
        pltpu.sync_copy(dst_hbm.at[wid], idx_v)
        pltpu.async_copy(ys_hbm.at[idx_v.at[0]], buf_v, sem).wait()
        pltpu.sync_copy(buf_v, y0_hbm.at[pl.ds(base, TPW)])
        pltpu.async_copy(ys_hbm.at[idx_v.at[1]], buf_v, sem).wait()
        pltpu.sync_copy(buf_v, y1_hbm.at[pl.ds(base, TPW)])

    return k(ys, dst3)


def _shared_kernel(x_ref, swg_ref, swu_ref, swd_ref, out_ref):
    out_ref[...] = _glu(x_ref[...], swg_ref[...], swu_ref[...], swd_ref[...])


def _shared(x, sWg, sWu, sWd):
    return pl.pallas_call(
        _shared_kernel,
        grid=(T // TB,),
        in_specs=[
            pl.BlockSpec((TB, D), lambda i: (i, 0)),
            pl.BlockSpec((SHARED_DFF, D), lambda i: (0, 0)),
            pl.BlockSpec((SHARED_DFF, D), lambda i: (0, 0)),
            pl.BlockSpec((D, SHARED_DFF), lambda i: (0, 0)),
        ],
        out_specs=pl.BlockSpec((TB, D), lambda i: (i, 0)),
        out_shape=jax.ShapeDtypeStruct((T, D), jnp.float32),
    )(x, sWg, sWu, sWd)


def _final_kernel(ysh_ref, y0_ref, y1_ref, w_ref, out_ref):
    w = w_ref[...]
    out_ref[...] = (ysh_ref[...] + w[:, 0:1] * y0_ref[...]
                    + w[:, 1:2] * y1_ref[...])


def _final(y_sh, y0, y1, wts):
    fb = 512
    return pl.pallas_call(
        _final_kernel,
        grid=(T // fb,),
        in_specs=[
            pl.BlockSpec((fb, D), lambda i: (i, 0)),
            pl.BlockSpec((fb, D), lambda i: (i, 0)),
            pl.BlockSpec((fb, D), lambda i: (i, 0)),
            pl.BlockSpec((fb, 2), lambda i: (i, 0)),
        ],
        out_specs=pl.BlockSpec((fb, D), lambda i: (i, 0)),
        out_shape=jax.ShapeDtypeStruct((T, D), jnp.float32),
    )(y_sh, y0, y1, wts)


def kernel(hidden_states, gate_weight, Wg, Wu, Wd, sWg, sWu, sWd):
    x = hidden_states
    dst3, wts, meta = _gate(x, gate_weight)
    xs = _sc_scatter(x, dst3)
    y_sh = _shared(x, sWg, sWu, sWd)
    ys = _grouped(meta, xs, Wg, Wu, Wd)
    y0, y1 = _sc_gather(ys, dst3)
    return _final(y_sh, y0, y1, wts)
